# baseline (speedup 1.0000x reference)
import functools

import jax
import jax.numpy as jnp
from jax import lax
from jax.experimental import pallas as pl
from jax.experimental.pallas import tpu as pltpu
from jax.experimental.pallas import tpu_sc as plsc

Z = 128
H = 128
N_NODES = 10000
N_EDGES = 320000

NC = 2
NS = 16
NW = NC * NS
EPW = N_EDGES // NW
CHUNK = 80
NCHUNK = EPW // CHUNK
UNROLL = 5

NPAD = 10240
NPT = NPAD // NS
GBUF = 8
GLOOK = 4
SBUF = 3

EBLK = 2000
NBLK = 1000

_mesh = plsc.VectorSubcoreMesh(
    core_axis_name="c", subcore_axis_name="s", num_cores=NC, num_subcores=NS
)


@functools.partial(
    pl.kernel,
    out_type=jax.ShapeDtypeStruct((N_EDGES, Z), jnp.float32),
    mesh=_mesh,
    scratch_types=[
        pltpu.VMEM((NCHUNK, CHUNK), jnp.int32),
        pltpu.VMEM((NCHUNK, CHUNK), jnp.int32),
        pltpu.VMEM((GBUF, CHUNK, Z), jnp.float32),
        pltpu.SemaphoreType.DMA((GBUF,)),
        pltpu.SemaphoreType.DMA((GBUF,)),
    ],
)
def _gather_add(ps_hbm, pr_hbm, s_hbm, r_hbm, out_hbm, idx_s, idx_r, rows,
                sem_g, sem_w):
    wid = lax.axis_index("s") * NC + lax.axis_index("c")
    base = wid * EPW
    pltpu.sync_copy(s_hbm.at[wid], idx_s)
    pltpu.sync_copy(r_hbm.at[wid], idx_r)

    def ps_copy(j, b):
        return pltpu.make_async_copy(ps_hbm.at[idx_s.at[j]], rows.at[b],
                                     sem_g.at[b])

    def pr_copy(j, b):
        return pltpu.make_async_copy(pr_hbm.at[idx_r.at[j]], rows.at[b],
                                     sem_g.at[b])

    def w_copy(j, b):
        return pltpu.make_async_copy(
            rows.at[b], out_hbm.at[pl.ds(base + j * CHUNK, CHUNK)], sem_w.at[b]
        )

    for t in range(GLOOK):
        ps_copy(t, t).start()

    def chunk_step(j):
        b = lax.rem(j, GBUF)
        ps_copy(j, b).wait()
        pr_copy(j, b).start(add=True)
        nj = j + GLOOK
        nb = lax.rem(nj, GBUF)

        @pl.when(nj < NCHUNK)
        def _():
            @pl.when(nj >= GBUF)
            def _():
                w_copy(nj - GBUF, nb).wait()

            ps_copy(nj, nb).start()

        pr_copy(j, b).wait()
        w_copy(j, b).start()

    def body(jj, carry):
        for u in range(UNROLL):
            chunk_step(jj * UNROLL + u)
        return carry

    lax.fori_loop(0, NCHUNK // UNROLL, body, 0)
    for t in range(GBUF):
        k = NCHUNK - GBUF + t
        w_copy(k, k % GBUF).wait()


@functools.partial(
    pl.kernel,
    out_type=jax.ShapeDtypeStruct((NC, NPAD, Z), jnp.float32),
    mesh=_mesh,
    scratch_types=[
        pltpu.VMEM((NCHUNK, CHUNK), jnp.int32),
        pltpu.VMEM((SBUF, CHUNK, Z), jnp.float32),
        pltpu.VMEM_SHARED((NPAD, Z), jnp.float32),
        pltpu.SemaphoreType.DMA((SBUF,)),
        pltpu.SemaphoreType.DMA((SBUF,)),
    ],
)
def _scatter_add(upd_hbm, r_hbm, out_hbm, idx_r, rows, acc, sem_l, sem_s):
    c = lax.axis_index("c")
    s = lax.axis_index("s")
    wid = s * NC + c
    base = wid * EPW

    zvec = jnp.zeros((16,), jnp.float32)
    stage = rows.at[0]

    def zrow(i, carry):
        for k in range(Z // 16):
            stage[i, pl.ds(k * 16, 16)] = zvec
        return carry

    lax.fori_loop(0, CHUNK, zrow, 0)
    for q in range(NPT // CHUNK):
        pltpu.sync_copy(stage, acc.at[pl.ds(s * NPT + q * CHUNK, CHUNK)])
    pltpu.sync_copy(r_hbm.at[wid], idx_r)
    plsc.subcore_barrier()

    def l_copy(j, b):
        return pltpu.make_async_copy(
            upd_hbm.at[pl.ds(base + j * CHUNK, CHUNK)], rows.at[b], sem_l.at[b]
        )

    def s_copy(j, b):
        return pltpu.make_async_copy(rows.at[b], acc.at[idx_r.at[j]],
                                     sem_s.at[b])

    l_copy(0, 0).start()

    def chunk_step(j):
        b = lax.rem(j, SBUF)
        l_copy(j, b).wait()
        nj = j + 1
        nb = lax.rem(nj, SBUF)

        @pl.when(nj < NCHUNK)
        def _():
            @pl.when(nj >= SBUF)
            def _():
                s_copy(nj - SBUF, nb).wait()

            l_copy(nj, nb).start()

        s_copy(j, b).start(add=True)

    def body(jj, carry):
        for u in range(UNROLL):
            chunk_step(jj * UNROLL + u)
        return carry

    lax.fori_loop(0, NCHUNK // UNROLL, body, 0)
    for t in range(SBUF):
        k = NCHUNK - SBUF + t
        s_copy(k, k % SBUF).wait()
    plsc.subcore_barrier()

    for q in range(NPT // CHUNK):
        off = s * NPT + q * CHUNK
        pltpu.sync_copy(acc.at[pl.ds(off, CHUNK)], stage)
        pltpu.sync_copy(stage, out_hbm.at[c].at[pl.ds(off, CHUNK)])


def _proj_body(nf_ref, w1a_ref, w1b_ref, b1_ref, ps_ref, pr_ref):
    nf = nf_ref[...]
    ps_ref[...] = (
        jnp.dot(nf, w1a_ref[...], preferred_element_type=jnp.float32) + b1_ref[...]
    )
    pr_ref[...] = jnp.dot(nf, w1b_ref[...], preferred_element_type=jnp.float32)


_proj = pl.pallas_call(
    _proj_body,
    grid=(N_NODES // NBLK,),
    in_specs=[
        pl.BlockSpec((NBLK, Z), lambda i: (i, 0)),
        pl.BlockSpec((Z, H), lambda i: (0, 0)),
        pl.BlockSpec((Z, H), lambda i: (0, 0)),
        pl.BlockSpec((1, H), lambda i: (0, 0)),
    ],
    out_specs=[
        pl.BlockSpec((NBLK, H), lambda i: (i, 0)),
        pl.BlockSpec((NBLK, H), lambda i: (i, 0)),
    ],
    out_shape=[
        jax.ShapeDtypeStruct((N_NODES, H), jnp.float32),
        jax.ShapeDtypeStruct((N_NODES, H), jnp.float32),
    ],
)


def _edge_body(g_ref, e_ref, w1c, w2, b2, w3, b3, upd_ref, new_ref):
    e = e_ref[...]
    h1 = jnp.maximum(
        g_ref[...] + jnp.dot(e, w1c[...], preferred_element_type=jnp.float32), 0.0
    )
    h2 = jnp.maximum(
        jnp.dot(h1, w2[...], preferred_element_type=jnp.float32) + b2[...], 0.0
    )
    upd = jnp.dot(h2, w3[...], preferred_element_type=jnp.float32) + b3[...]
    upd_ref[...] = upd
    new_ref[...] = upd + e


_edge_mlp = pl.pallas_call(
    _edge_body,
    grid=(N_EDGES // EBLK,),
    in_specs=[
        pl.BlockSpec((EBLK, H), lambda i: (i, 0)),
        pl.BlockSpec((EBLK, Z), lambda i: (i, 0)),
        pl.BlockSpec((Z, H), lambda i: (0, 0)),
        pl.BlockSpec((H, H), lambda i: (0, 0)),
        pl.BlockSpec((1, H), lambda i: (0, 0)),
        pl.BlockSpec((H, Z), lambda i: (0, 0)),
        pl.BlockSpec((1, Z), lambda i: (0, 0)),
    ],
    out_specs=[
        pl.BlockSpec((EBLK, Z), lambda i: (i, 0)),
        pl.BlockSpec((EBLK, Z), lambda i: (i, 0)),
    ],
    out_shape=[
        jax.ShapeDtypeStruct((N_EDGES, Z), jnp.float32),
        jax.ShapeDtypeStruct((N_EDGES, Z), jnp.float32),
    ],
)


def _node_body(nf_ref, p0, p1, w1a, w1b, b1, w2, b2, w3, b3, out_ref):
    nf = nf_ref[...]
    agg = p0[...] + p1[...]
    h1 = jnp.maximum(
        jnp.dot(nf, w1a[...], preferred_element_type=jnp.float32)
        + jnp.dot(agg, w1b[...], preferred_element_type=jnp.float32)
        + b1[...],
        0.0,
    )
    h2 = jnp.maximum(
        jnp.dot(h1, w2[...], preferred_element_type=jnp.float32) + b2[...], 0.0
    )
    out_ref[...] = (
        jnp.dot(h2, w3[...], preferred_element_type=jnp.float32) + b3[...] + nf
    )


_node_mlp = pl.pallas_call(
    _node_body,
    grid=(N_NODES // NBLK,),
    in_specs=[
        pl.BlockSpec((NBLK, Z), lambda i: (i, 0)),
        pl.BlockSpec((NBLK, Z), lambda i: (i, 0)),
        pl.BlockSpec((NBLK, Z), lambda i: (i, 0)),
        pl.BlockSpec((Z, H), lambda i: (0, 0)),
        pl.BlockSpec((Z, H), lambda i: (0, 0)),
        pl.BlockSpec((1, H), lambda i: (0, 0)),
        pl.BlockSpec((H, H), lambda i: (0, 0)),
        pl.BlockSpec((1, H), lambda i: (0, 0)),
        pl.BlockSpec((H, Z), lambda i: (0, 0)),
        pl.BlockSpec((1, Z), lambda i: (0, 0)),
    ],
    out_specs=pl.BlockSpec((NBLK, Z), lambda i: (i, 0)),
    out_shape=jax.ShapeDtypeStruct((N_NODES, Z), jnp.float32),
)


def kernel(node_features, edge_features, senders, receivers,
           eW1, eb1, eW2, eb2, eW3, eb3,
           nW1, nb1, nW2, nb2, nW3, nb3):
    s32 = senders.astype(jnp.int32).reshape(NW, NCHUNK, CHUNK)
    r32 = receivers.astype(jnp.int32).reshape(NW, NCHUNK, CHUNK)

    ps, pr = _proj(node_features, eW1[:Z], eW1[Z:2 * Z], eb1.reshape(1, H))
    g = _gather_add(ps, pr, s32, r32)
    upd, new_edge = _edge_mlp(
        g, edge_features, eW1[2 * Z:], eW2, eb2.reshape(1, H), eW3,
        eb3.reshape(1, Z),
    )
    parts = _scatter_add(upd, r32)
    new_node = _node_mlp(
        node_features, parts[0, :N_NODES], parts[1, :N_NODES],
        nW1[:Z], nW1[Z:], nb1.reshape(1, H), nW2, nb2.reshape(1, H), nW3,
        nb3.reshape(1, Z),
    )
    return new_node, new_edge


# speedup vs baseline: 1.0670x; 1.0670x over previous
import functools

import jax
import jax.numpy as jnp
from jax import lax
from jax.experimental import pallas as pl
from jax.experimental.pallas import tpu as pltpu
from jax.experimental.pallas import tpu_sc as plsc

Z = 128
H = 128
N_NODES = 10000
N_EDGES = 320000

NC = 2
NS = 16
NW = NC * NS

K = 2
NE_K = N_EDGES // K
EPW_K = NE_K // NW
SUPER = 200
SUBT = 2
SUBC = SUPER // SUBT
NSUP = EPW_K // SUPER
GBUF = 4
GUNROLL = 5

EPW = N_EDGES // NW
CHUNK = 80
NCHUNK = EPW // CHUNK
UNROLL = 5
NPAD = 10240
NPT = NPAD // NS
SBUF = 2

EBLK = 2000
NBLK = 1000

_mesh = plsc.VectorSubcoreMesh(
    core_axis_name="c", subcore_axis_name="s", num_cores=NC, num_subcores=NS
)


def _gather_body(ps_hbm, pr_hbm, s_hbm, r_hbm, out_hbm, idx_s, idx_r, rows,
                 sem_g, sem_w):
    wid = lax.axis_index("s") * NC + lax.axis_index("c")
    base = wid * EPW_K
    pltpu.sync_copy(s_hbm.at[wid], idx_s)
    pltpu.sync_copy(r_hbm.at[wid], idx_r)

    def ps_copy(j, q, b):
        return pltpu.make_async_copy(
            ps_hbm.at[idx_s.at[j, q]],
            rows.at[b, pl.ds(q * SUBC, SUBC)], sem_g.at[b])

    def pr_copy(j, q, b):
        return pltpu.make_async_copy(
            pr_hbm.at[idx_r.at[j, q]],
            rows.at[b, pl.ds(q * SUBC, SUBC)], sem_g.at[b])

    def w_copy(j, b):
        return pltpu.make_async_copy(
            rows.at[b], out_hbm.at[pl.ds(base + j * SUPER, SUPER)], sem_w.at[b]
        )

    def ps_start(j, b):
        for q in range(SUBT):
            ps_copy(j, q, b).start()

    ps_start(0, 0)
    ps_start(1, 1)

    def chunk_step(j):
        b = lax.rem(j, GBUF)
        for q in range(SUBT):
            ps_copy(j, q, b).wait()
        for q in range(SUBT):
            pr_copy(j, q, b).start(add=True)
        nj = j + 2
        nb = lax.rem(nj, GBUF)

        @pl.when(nj < NSUP)
        def _():
            @pl.when(nj >= GBUF)
            def _():
                w_copy(nj - GBUF, nb).wait()

            ps_start(nj, nb)

        for q in range(SUBT):
            pr_copy(j, q, b).wait()
        w_copy(j, b).start()

    def body(jj, carry):
        for u in range(GUNROLL):
            chunk_step(jj * GUNROLL + u)
        return carry

    lax.fori_loop(0, NSUP // GUNROLL, body, 0)
    for t in range(GBUF):
        k = NSUP - GBUF + t
        w_copy(k, k % GBUF).wait()


_gather_scratch = [
    pltpu.VMEM((NSUP, SUBT, SUBC), jnp.int32),
    pltpu.VMEM((NSUP, SUBT, SUBC), jnp.int32),
    pltpu.VMEM((GBUF, SUPER, Z), jnp.float32),
    pltpu.SemaphoreType.DMA((GBUF,)),
    pltpu.SemaphoreType.DMA((GBUF,)),
]

_gather0 = pl.kernel(
    lambda ps, pr, s, r, out, dummy, *scr: _gather_body(ps, pr, s, r, out, *scr),
    out_type=(
        jax.ShapeDtypeStruct((NE_K, Z), jnp.float32),
        jax.ShapeDtypeStruct((N_EDGES, Z), jnp.float32),
    ),
    mesh=_mesh,
    scratch_types=_gather_scratch,
)

_gather1 = pl.kernel(
    _gather_body,
    out_type=jax.ShapeDtypeStruct((NE_K, Z), jnp.float32),
    mesh=_mesh,
    scratch_types=_gather_scratch,
)


@functools.partial(
    pl.kernel,
    out_type=jax.ShapeDtypeStruct((NC, NPAD, Z), jnp.float32),
    mesh=_mesh,
    scratch_types=[
        pltpu.VMEM((NCHUNK, CHUNK), jnp.int32),
        pltpu.VMEM((SBUF, CHUNK, Z), jnp.float32),
        pltpu.VMEM_SHARED((NPAD, Z), jnp.float32),
        pltpu.SemaphoreType.DMA((SBUF,)),
        pltpu.SemaphoreType.DMA((SBUF,)),
    ],
)
def _scatter_add(upd0_hbm, upd1_hbm, r_hbm, out_hbm, idx_r, rows, acc,
                 sem_l, sem_s):
    c = lax.axis_index("c")
    s = lax.axis_index("s")
    wid = s * NC + c

    zvec = jnp.zeros((16,), jnp.float32)
    stage = rows.at[0]

    def zrow(i, carry):
        for k in range(Z // 16):
            stage[i, pl.ds(k * 16, 16)] = zvec
        return carry

    lax.fori_loop(0, CHUNK, zrow, 0)
    for q in range(NPT // CHUNK):
        pltpu.sync_copy(stage, acc.at[pl.ds(s * NPT + q * CHUNK, CHUNK)])
    pltpu.sync_copy(r_hbm.at[wid], idx_r)
    plsc.subcore_barrier()

    def run(upd_hbm, base):
        def l_copy(j, b):
            return pltpu.make_async_copy(
                upd_hbm.at[pl.ds(base + j * CHUNK, CHUNK)], rows.at[b],
                sem_l.at[b])

        def s_copy(j, b):
            return pltpu.make_async_copy(rows.at[b], acc.at[idx_r.at[j]],
                                         sem_s.at[b])

        l_copy(0, 0).start()

        def chunk_step(j):
            b = lax.rem(j, SBUF)
            l_copy(j, b).wait()
            nj = j + 1
            nb = lax.rem(nj, SBUF)

            @pl.when(nj < NCHUNK)
            def _():
                @pl.when(nj >= SBUF)
                def _():
                    s_copy(nj - SBUF, nb).wait()

                l_copy(nj, nb).start()

            s_copy(j, b).start(add=True)

        def body(jj, carry):
            for u in range(UNROLL):
                chunk_step(jj * UNROLL + u)
            return carry

        lax.fori_loop(0, NCHUNK // UNROLL, body, 0)
        for t in range(SBUF):
            k = NCHUNK - SBUF + t
            s_copy(k, k % SBUF).wait()

    @pl.when(wid < NW // 2)
    def _():
        run(upd0_hbm, wid * EPW)

    @pl.when(wid >= NW // 2)
    def _():
        run(upd1_hbm, wid * EPW - NE_K)

    plsc.subcore_barrier()

    for q in range(NPT // CHUNK):
        off = s * NPT + q * CHUNK
        pltpu.sync_copy(acc.at[pl.ds(off, CHUNK)], stage)
        pltpu.sync_copy(stage, out_hbm.at[c].at[pl.ds(off, CHUNK)])


def _proj_body(nf_ref, w1a_ref, w1b_ref, b1_ref, ps_ref, pr_ref):
    nf = nf_ref[...]
    ps_ref[...] = (
        jnp.dot(nf, w1a_ref[...], preferred_element_type=jnp.float32) + b1_ref[...]
    )
    pr_ref[...] = jnp.dot(nf, w1b_ref[...], preferred_element_type=jnp.float32)


_proj = pl.pallas_call(
    _proj_body,
    grid=(N_NODES // NBLK,),
    in_specs=[
        pl.BlockSpec((NBLK, Z), lambda i: (i, 0)),
        pl.BlockSpec((Z, H), lambda i: (0, 0)),
        pl.BlockSpec((Z, H), lambda i: (0, 0)),
        pl.BlockSpec((1, H), lambda i: (0, 0)),
    ],
    out_specs=[
        pl.BlockSpec((NBLK, H), lambda i: (i, 0)),
        pl.BlockSpec((NBLK, H), lambda i: (i, 0)),
    ],
    out_shape=[
        jax.ShapeDtypeStruct((N_NODES, H), jnp.float32),
        jax.ShapeDtypeStruct((N_NODES, H), jnp.float32),
    ],
)


def _edge_body(g_ref, e_ref, w1c, w2, b2, w3, b3, newin_ref, upd_ref, new_ref):
    del newin_ref
    e = e_ref[...]
    h1 = jnp.maximum(
        g_ref[...] + jnp.dot(e, w1c[...], preferred_element_type=jnp.float32), 0.0
    )
    h2 = jnp.maximum(
        jnp.dot(h1, w2[...], preferred_element_type=jnp.float32) + b2[...], 0.0
    )
    upd = jnp.dot(h2, w3[...], preferred_element_type=jnp.float32) + b3[...]
    upd_ref[...] = upd
    new_ref[...] = upd + e


def _make_edge_mlp(off_blocks):
    return pl.pallas_call(
        _edge_body,
        grid=(NE_K // EBLK,),
        in_specs=[
            pl.BlockSpec((EBLK, H), lambda i: (i, 0)),
            pl.BlockSpec((EBLK, Z), lambda i: (i + off_blocks, 0)),
            pl.BlockSpec((Z, H), lambda i: (0, 0)),
            pl.BlockSpec((H, H), lambda i: (0, 0)),
            pl.BlockSpec((1, H), lambda i: (0, 0)),
            pl.BlockSpec((H, Z), lambda i: (0, 0)),
            pl.BlockSpec((1, Z), lambda i: (0, 0)),
            pl.BlockSpec((8, Z), lambda i: (0, 0)),
        ],
        out_specs=[
            pl.BlockSpec((EBLK, Z), lambda i: (i, 0)),
            pl.BlockSpec((EBLK, Z), lambda i: (i + off_blocks, 0)),
        ],
        out_shape=[
            jax.ShapeDtypeStruct((NE_K, Z), jnp.float32),
            jax.ShapeDtypeStruct((N_EDGES, Z), jnp.float32),
        ],
        input_output_aliases={7: 1},
    )


_edge_mlps = [_make_edge_mlp(k * (NE_K // EBLK)) for k in range(K)]


def _node_body(nf_ref, p0, p1, w1a, w1b, b1, w2, b2, w3, b3, out_ref):
    nf = nf_ref[...]
    agg = p0[...] + p1[...]
    h1 = jnp.maximum(
        jnp.dot(nf, w1a[...], preferred_element_type=jnp.float32)
        + jnp.dot(agg, w1b[...], preferred_element_type=jnp.float32)
        + b1[...],
        0.0,
    )
    h2 = jnp.maximum(
        jnp.dot(h1, w2[...], preferred_element_type=jnp.float32) + b2[...], 0.0
    )
    out_ref[...] = (
        jnp.dot(h2, w3[...], preferred_element_type=jnp.float32) + b3[...] + nf
    )


_node_mlp = pl.pallas_call(
    _node_body,
    grid=(N_NODES // NBLK,),
    in_specs=[
        pl.BlockSpec((NBLK, Z), lambda i: (i, 0)),
        pl.BlockSpec((NBLK, Z), lambda i: (i, 0)),
        pl.BlockSpec((NBLK, Z), lambda i: (i, 0)),
        pl.BlockSpec((Z, H), lambda i: (0, 0)),
        pl.BlockSpec((Z, H), lambda i: (0, 0)),
        pl.BlockSpec((1, H), lambda i: (0, 0)),
        pl.BlockSpec((H, H), lambda i: (0, 0)),
        pl.BlockSpec((1, H), lambda i: (0, 0)),
        pl.BlockSpec((H, Z), lambda i: (0, 0)),
        pl.BlockSpec((1, Z), lambda i: (0, 0)),
    ],
    out_specs=pl.BlockSpec((NBLK, Z), lambda i: (i, 0)),
    out_shape=jax.ShapeDtypeStruct((N_NODES, Z), jnp.float32),
)


def kernel(node_features, edge_features, senders, receivers,
           eW1, eb1, eW2, eb2, eW3, eb3,
           nW1, nb1, nW2, nb2, nW3, nb3):
    s32 = senders.astype(jnp.int32).reshape(K, NW, NSUP, SUBT, SUBC)
    r32g = receivers.astype(jnp.int32).reshape(K, NW, NSUP, SUBT, SUBC)
    r32s = receivers.astype(jnp.int32).reshape(NW, NCHUNK, CHUNK)

    ps, pr = _proj(node_features, eW1[:Z], eW1[Z:2 * Z], eb1.reshape(1, H))

    w1c = eW1[2 * Z:]
    b2r = eb2.reshape(1, H)
    b3r = eb3.reshape(1, Z)

    g0, new_buf = _gather0(ps, pr, s32[0], r32g[0])
    upd0, new_buf = _edge_mlps[0](
        g0, edge_features, w1c, eW2, b2r, eW3, b3r, new_buf
    )
    g1 = _gather1(ps, pr, s32[1], r32g[1])
    upd1, new_edge = _edge_mlps[1](
        g1, edge_features, w1c, eW2, b2r, eW3, b3r, new_buf
    )

    parts = _scatter_add(upd0, upd1, r32s)
    new_node = _node_mlp(
        node_features, parts[0, :N_NODES], parts[1, :N_NODES],
        nW1[:Z], nW1[Z:], nb1.reshape(1, H), nW2, nb2.reshape(1, H), nW3,
        nb3.reshape(1, Z),
    )
    return new_node, new_edge


# speedup vs baseline: 1.1732x; 1.0996x over previous
import functools

import jax
import jax.numpy as jnp
from jax import lax
from jax.experimental import pallas as pl
from jax.experimental.pallas import tpu as pltpu
from jax.experimental.pallas import tpu_sc as plsc

Z = 128
H = 128
N_NODES = 10000
N_EDGES = 320000

NC = 2
NS = 16
NW = NC * NS

K = 2
NE_K = N_EDGES // K
EPW_K = NE_K // NW
SUPER = 200
SUBT = 2
SUBC = SUPER // SUBT
NSUP = EPW_K // SUPER
GBUF = 4
GUNROLL = 5

SCH = 120
NCH_M = 41
TAIL = 80
NPAD = 10240
NPT = NPAD // NS
SBUF = 2

EBLK = 2000
NBLK = 1000

_mesh = plsc.VectorSubcoreMesh(
    core_axis_name="c", subcore_axis_name="s", num_cores=NC, num_subcores=NS
)


def _gather_body(ps_hbm, pr_hbm, s_hbm, r_hbm, out_hbm, idx_s, idx_r, rows,
                 sem_g, sem_w):
    wid = lax.axis_index("s") * NC + lax.axis_index("c")
    base = wid * EPW_K
    pltpu.sync_copy(s_hbm.at[wid], idx_s)
    pltpu.sync_copy(r_hbm.at[wid], idx_r)

    def ps_copy(j, q, b):
        return pltpu.make_async_copy(
            ps_hbm.at[idx_s.at[j, q]],
            rows.at[b, pl.ds(q * SUBC, SUBC)], sem_g.at[b])

    def pr_copy(j, q, b):
        return pltpu.make_async_copy(
            pr_hbm.at[idx_r.at[j, q]],
            rows.at[b, pl.ds(q * SUBC, SUBC)], sem_g.at[b])

    def w_copy(j, b):
        return pltpu.make_async_copy(
            rows.at[b], out_hbm.at[pl.ds(base + j * SUPER, SUPER)], sem_w.at[b]
        )

    def ps_start(j, b):
        for q in range(SUBT):
            ps_copy(j, q, b).start()

    ps_start(0, 0)
    ps_start(1, 1)

    def chunk_step(j):
        b = lax.rem(j, GBUF)
        for q in range(SUBT):
            ps_copy(j, q, b).wait()
        for q in range(SUBT):
            pr_copy(j, q, b).start(add=True)
        nj = j + 2
        nb = lax.rem(nj, GBUF)

        @pl.when(nj < NSUP)
        def _():
            @pl.when(nj >= GBUF)
            def _():
                w_copy(nj - GBUF, nb).wait()

            ps_start(nj, nb)

        for q in range(SUBT):
            pr_copy(j, q, b).wait()
        w_copy(j, b).start()

    def body(jj, carry):
        for u in range(GUNROLL):
            chunk_step(jj * GUNROLL + u)
        return carry

    lax.fori_loop(0, NSUP // GUNROLL, body, 0)
    for t in range(GBUF):
        k = NSUP - GBUF + t
        w_copy(k, k % GBUF).wait()


_gather_scratch = [
    pltpu.VMEM((NSUP, SUBT, SUBC), jnp.int32),
    pltpu.VMEM((NSUP, SUBT, SUBC), jnp.int32),
    pltpu.VMEM((GBUF, SUPER, Z), jnp.float32),
    pltpu.SemaphoreType.DMA((GBUF,)),
    pltpu.SemaphoreType.DMA((GBUF,)),
]

_gather0 = pl.kernel(
    lambda ps, pr, s, r, out, dummy, *scr: _gather_body(ps, pr, s, r, out, *scr),
    out_type=(
        jax.ShapeDtypeStruct((NE_K, Z), jnp.float32),
        jax.ShapeDtypeStruct((N_EDGES, Z), jnp.float32),
    ),
    mesh=_mesh,
    scratch_types=_gather_scratch,
)

_gather1 = pl.kernel(
    _gather_body,
    out_type=jax.ShapeDtypeStruct((NE_K, Z), jnp.float32),
    mesh=_mesh,
    scratch_types=_gather_scratch,
)


@functools.partial(
    pl.kernel,
    out_type=jax.ShapeDtypeStruct((NC, NPAD, Z), jnp.float32),
    mesh=_mesh,
    scratch_types=[
        pltpu.VMEM((NCH_M, SCH), jnp.int32),
        pltpu.VMEM((1, TAIL), jnp.int32),
        pltpu.VMEM((SBUF, SCH, Z), jnp.float32),
        pltpu.VMEM_SHARED((NPAD, Z), jnp.float32),
        pltpu.SemaphoreType.DMA((SBUF,)),
        pltpu.SemaphoreType.DMA((SBUF,)),
    ],
)
def _scatter_add(upd_hbm, rm_hbm, rt_hbm, out_hbm, idx_m, idx_t, rows, acc,
                 sem_l, sem_s):
    c = lax.axis_index("c")
    s = lax.axis_index("s")
    wid = s * NC + c
    base = wid * EPW_K
    ntot = NCH_M + 1

    zvec = jnp.zeros((16,), jnp.float32)
    stage = rows.at[0]

    def zrow(i, carry):
        for k in range(Z // 16):
            stage[i, pl.ds(k * 16, 16)] = zvec
        return carry

    lax.fori_loop(0, SCH, zrow, 0)
    for q in range(NPT // SCH):
        pltpu.sync_copy(stage, acc.at[pl.ds(s * NPT + q * SCH, SCH)])
    pltpu.sync_copy(
        stage.at[pl.ds(0, NPT - (NPT // SCH) * SCH)],
        acc.at[pl.ds(s * NPT + (NPT // SCH) * SCH, NPT - (NPT // SCH) * SCH)],
    )
    pltpu.sync_copy(rm_hbm.at[wid], idx_m)
    pltpu.sync_copy(rt_hbm.at[wid], idx_t)
    plsc.subcore_barrier()

    def _is_tail(j):
        return isinstance(j, int) and j == NCH_M

    def l_copy(j, b):
        if _is_tail(j):
            return pltpu.make_async_copy(
                upd_hbm.at[pl.ds(base + NCH_M * SCH, TAIL)],
                rows.at[b, pl.ds(0, TAIL)], sem_l.at[b])
        return pltpu.make_async_copy(
            upd_hbm.at[pl.ds(base + j * SCH, SCH)], rows.at[b], sem_l.at[b])

    def s_copy(j, b):
        if _is_tail(j):
            return pltpu.make_async_copy(rows.at[b, pl.ds(0, TAIL)],
                                         acc.at[idx_t.at[0]], sem_s.at[b])
        return pltpu.make_async_copy(rows.at[b], acc.at[idx_m.at[j]],
                                     sem_s.at[b])

    l_copy(0, 0).start()

    def chunk_step(j, nj_static=None):
        b = j % SBUF if isinstance(j, int) else lax.rem(j, SBUF)
        l_copy(j, b).wait()
        nj = j + 1
        nb = nj % SBUF if isinstance(nj, int) else lax.rem(nj, SBUF)
        if isinstance(j, int):
            if nj < ntot:
                if nj >= SBUF:
                    s_copy(nj - SBUF, nb).wait()
                l_copy(nj, nb).start()
        else:
            @pl.when(nj >= SBUF)
            def _():
                s_copy(nj - SBUF, nb).wait()

            l_copy(nj, nb).start()
        s_copy(j, b).start(add=True)

    def body(j, carry):
        chunk_step(j)
        return carry

    lax.fori_loop(0, NCH_M - 1, body, 0)
    chunk_step(NCH_M - 1)
    chunk_step(NCH_M)
    for t in range(SBUF):
        k = ntot - SBUF + t
        s_copy(k, k % SBUF).wait()
    plsc.subcore_barrier()

    for q in range(NPT // SCH):
        off = s * NPT + q * SCH
        pltpu.sync_copy(acc.at[pl.ds(off, SCH)], stage)
        pltpu.sync_copy(stage, out_hbm.at[c].at[pl.ds(off, SCH)])
    rem = NPT - (NPT // SCH) * SCH
    off = s * NPT + (NPT // SCH) * SCH
    pltpu.sync_copy(acc.at[pl.ds(off, rem)], stage.at[pl.ds(0, rem)])
    pltpu.sync_copy(stage.at[pl.ds(0, rem)], out_hbm.at[c].at[pl.ds(off, rem)])


def _proj_body(nf_ref, w1a_ref, w1b_ref, b1_ref, ps_ref, pr_ref):
    nf = nf_ref[...]
    ps_ref[...] = (
        jnp.dot(nf, w1a_ref[...], preferred_element_type=jnp.float32) + b1_ref[...]
    )
    pr_ref[...] = jnp.dot(nf, w1b_ref[...], preferred_element_type=jnp.float32)


_proj = pl.pallas_call(
    _proj_body,
    grid=(N_NODES // NBLK,),
    in_specs=[
        pl.BlockSpec((NBLK, Z), lambda i: (i, 0)),
        pl.BlockSpec((Z, H), lambda i: (0, 0)),
        pl.BlockSpec((Z, H), lambda i: (0, 0)),
        pl.BlockSpec((1, H), lambda i: (0, 0)),
    ],
    out_specs=[
        pl.BlockSpec((NBLK, H), lambda i: (i, 0)),
        pl.BlockSpec((NBLK, H), lambda i: (i, 0)),
    ],
    out_shape=[
        jax.ShapeDtypeStruct((N_NODES, H), jnp.float32),
        jax.ShapeDtypeStruct((N_NODES, H), jnp.float32),
    ],
)


def _edge_body(g_ref, e_ref, w1c, w2, b2, w3, b3, newin_ref, upd_ref, new_ref):
    del newin_ref
    e = e_ref[...]
    h1 = jnp.maximum(
        g_ref[...] + jnp.dot(e, w1c[...], preferred_element_type=jnp.float32), 0.0
    )
    h2 = jnp.maximum(
        jnp.dot(h1, w2[...], preferred_element_type=jnp.float32) + b2[...], 0.0
    )
    upd = jnp.dot(h2, w3[...], preferred_element_type=jnp.float32) + b3[...]
    upd_ref[...] = upd
    new_ref[...] = upd + e


def _make_edge_mlp(off_blocks):
    return pl.pallas_call(
        _edge_body,
        grid=(NE_K // EBLK,),
        in_specs=[
            pl.BlockSpec((EBLK, H), lambda i: (i, 0)),
            pl.BlockSpec((EBLK, Z), lambda i: (i + off_blocks, 0)),
            pl.BlockSpec((Z, H), lambda i: (0, 0)),
            pl.BlockSpec((H, H), lambda i: (0, 0)),
            pl.BlockSpec((1, H), lambda i: (0, 0)),
            pl.BlockSpec((H, Z), lambda i: (0, 0)),
            pl.BlockSpec((1, Z), lambda i: (0, 0)),
            pl.BlockSpec((8, Z), lambda i: (0, 0)),
        ],
        out_specs=[
            pl.BlockSpec((EBLK, Z), lambda i: (i, 0)),
            pl.BlockSpec((EBLK, Z), lambda i: (i + off_blocks, 0)),
        ],
        out_shape=[
            jax.ShapeDtypeStruct((NE_K, Z), jnp.float32),
            jax.ShapeDtypeStruct((N_EDGES, Z), jnp.float32),
        ],
        input_output_aliases={7: 1},
    )


_edge_mlps = [_make_edge_mlp(k * (NE_K // EBLK)) for k in range(K)]


def _node_body(nf_ref, p0, p1, p2, p3, w1a, w1b, b1, w2, b2, w3, b3, out_ref):
    nf = nf_ref[...]
    agg = (p0[...] + p1[...]) + (p2[...] + p3[...])
    h1 = jnp.maximum(
        jnp.dot(nf, w1a[...], preferred_element_type=jnp.float32)
        + jnp.dot(agg, w1b[...], preferred_element_type=jnp.float32)
        + b1[...],
        0.0,
    )
    h2 = jnp.maximum(
        jnp.dot(h1, w2[...], preferred_element_type=jnp.float32) + b2[...], 0.0
    )
    out_ref[...] = (
        jnp.dot(h2, w3[...], preferred_element_type=jnp.float32) + b3[...] + nf
    )


_node_mlp = pl.pallas_call(
    _node_body,
    grid=(N_NODES // NBLK,),
    in_specs=[
        pl.BlockSpec((NBLK, Z), lambda i: (i, 0)),
        pl.BlockSpec((NBLK, Z), lambda i: (i, 0)),
        pl.BlockSpec((NBLK, Z), lambda i: (i, 0)),
        pl.BlockSpec((NBLK, Z), lambda i: (i, 0)),
        pl.BlockSpec((NBLK, Z), lambda i: (i, 0)),
        pl.BlockSpec((Z, H), lambda i: (0, 0)),
        pl.BlockSpec((Z, H), lambda i: (0, 0)),
        pl.BlockSpec((1, H), lambda i: (0, 0)),
        pl.BlockSpec((H, H), lambda i: (0, 0)),
        pl.BlockSpec((1, H), lambda i: (0, 0)),
        pl.BlockSpec((H, Z), lambda i: (0, 0)),
        pl.BlockSpec((1, Z), lambda i: (0, 0)),
    ],
    out_specs=pl.BlockSpec((NBLK, Z), lambda i: (i, 0)),
    out_shape=jax.ShapeDtypeStruct((N_NODES, Z), jnp.float32),
)


def kernel(node_features, edge_features, senders, receivers,
           eW1, eb1, eW2, eb2, eW3, eb3,
           nW1, nb1, nW2, nb2, nW3, nb3):
    s32 = senders.astype(jnp.int32).reshape(K, NW, NSUP, SUBT, SUBC)
    r32g = receivers.astype(jnp.int32).reshape(K, NW, NSUP, SUBT, SUBC)
    r32w = receivers.astype(jnp.int32).reshape(K, NW, EPW_K)
    r32m = r32w[:, :, :NCH_M * SCH].reshape(K, NW, NCH_M, SCH)
    r32t = r32w[:, :, NCH_M * SCH:].reshape(K, NW, 1, TAIL)

    ps, pr = _proj(node_features, eW1[:Z], eW1[Z:2 * Z], eb1.reshape(1, H))

    w1c = eW1[2 * Z:]
    b2r = eb2.reshape(1, H)
    b3r = eb3.reshape(1, Z)

    g0, new_buf = _gather0(ps, pr, s32[0], r32g[0])
    upd0, new_buf = _edge_mlps[0](
        g0, edge_features, w1c, eW2, b2r, eW3, b3r, new_buf
    )
    g1 = _gather1(ps, pr, s32[1], r32g[1])
    parts0 = _scatter_add(upd0, r32m[0], r32t[0])
    upd1, new_edge = _edge_mlps[1](
        g1, edge_features, w1c, eW2, b2r, eW3, b3r, new_buf
    )
    parts1 = _scatter_add(upd1, r32m[1], r32t[1])

    new_node = _node_mlp(
        node_features,
        parts0[0, :N_NODES], parts0[1, :N_NODES],
        parts1[0, :N_NODES], parts1[1, :N_NODES],
        nW1[:Z], nW1[Z:], nb1.reshape(1, H), nW2, nb2.reshape(1, H), nW3,
        nb3.reshape(1, Z),
    )
    return new_node, new_edge


# speedup vs baseline: 1.2054x; 1.0274x over previous
import functools

import jax
import jax.numpy as jnp
from jax import lax
from jax.experimental import pallas as pl
from jax.experimental.pallas import tpu as pltpu
from jax.experimental.pallas import tpu_sc as plsc

Z = 128
H = 128
N_NODES = 10000
N_EDGES = 320000

NC = 2
NS = 16
NW = NC * NS

K = 2
NE_K = N_EDGES // K
EPW_K = NE_K // NW
SUPER = 200
SUBT = 2
SUBC = SUPER // SUBT
NSUP = EPW_K // SUPER
GBUF = 4
GUNROLL = 5

SCH = 120
NCH_M = 41
TAIL = 80
NPAD = 10240
NPT = NPAD // NS
SBUF = 2

EBLK = 4000
NBLK = 1000

_mesh = plsc.VectorSubcoreMesh(
    core_axis_name="c", subcore_axis_name="s", num_cores=NC, num_subcores=NS
)


def _gather_body(ps_hbm, pr_hbm, s_hbm, r_hbm, out_hbm, idx_s, idx_r, rows,
                 sem_g, sem_w):
    wid = lax.axis_index("s") * NC + lax.axis_index("c")
    base = wid * EPW_K
    pltpu.sync_copy(s_hbm.at[wid], idx_s)
    pltpu.sync_copy(r_hbm.at[wid], idx_r)

    def ps_copy(j, q, b):
        return pltpu.make_async_copy(
            ps_hbm.at[idx_s.at[j, q]],
            rows.at[b, pl.ds(q * SUBC, SUBC)], sem_g.at[b])

    def pr_copy(j, q, b):
        return pltpu.make_async_copy(
            pr_hbm.at[idx_r.at[j, q]],
            rows.at[b, pl.ds(q * SUBC, SUBC)], sem_g.at[b])

    def w_copy(j, b):
        return pltpu.make_async_copy(
            rows.at[b], out_hbm.at[pl.ds(base + j * SUPER, SUPER)], sem_w.at[b]
        )

    def ps_start(j, b):
        for q in range(SUBT):
            ps_copy(j, q, b).start()

    ps_start(0, 0)
    ps_start(1, 1)

    def chunk_step(j):
        b = lax.rem(j, GBUF)
        for q in range(SUBT):
            ps_copy(j, q, b).wait()
        for q in range(SUBT):
            pr_copy(j, q, b).start(add=True)
        nj = j + 2
        nb = lax.rem(nj, GBUF)

        @pl.when(nj < NSUP)
        def _():
            @pl.when(nj >= GBUF)
            def _():
                w_copy(nj - GBUF, nb).wait()

            ps_start(nj, nb)

        for q in range(SUBT):
            pr_copy(j, q, b).wait()
        w_copy(j, b).start()

    def body(jj, carry):
        for u in range(GUNROLL):
            chunk_step(jj * GUNROLL + u)
        return carry

    lax.fori_loop(0, NSUP // GUNROLL, body, 0)
    for t in range(GBUF):
        k = NSUP - GBUF + t
        w_copy(k, k % GBUF).wait()


_gather_scratch = [
    pltpu.VMEM((NSUP, SUBT, SUBC), jnp.int32),
    pltpu.VMEM((NSUP, SUBT, SUBC), jnp.int32),
    pltpu.VMEM((GBUF, SUPER, Z), jnp.float32),
    pltpu.SemaphoreType.DMA((GBUF,)),
    pltpu.SemaphoreType.DMA((GBUF,)),
]

_gather0 = pl.kernel(
    lambda ps, pr, s, r, out, dummy, *scr: _gather_body(ps, pr, s, r, out, *scr),
    out_type=(
        jax.ShapeDtypeStruct((NE_K, Z), jnp.float32),
        jax.ShapeDtypeStruct((N_EDGES, Z), jnp.float32),
    ),
    mesh=_mesh,
    scratch_types=_gather_scratch,
)

_gather1 = pl.kernel(
    _gather_body,
    out_type=jax.ShapeDtypeStruct((NE_K, Z), jnp.float32),
    mesh=_mesh,
    scratch_types=_gather_scratch,
)


@functools.partial(
    pl.kernel,
    out_type=jax.ShapeDtypeStruct((NC, NPAD, Z), jnp.float32),
    mesh=_mesh,
    scratch_types=[
        pltpu.VMEM((NCH_M, SCH), jnp.int32),
        pltpu.VMEM((1, TAIL), jnp.int32),
        pltpu.VMEM((SBUF, SCH, Z), jnp.float32),
        pltpu.VMEM_SHARED((NPAD, Z), jnp.float32),
        pltpu.SemaphoreType.DMA((SBUF,)),
        pltpu.SemaphoreType.DMA((SBUF,)),
    ],
)
def _scatter_add(upd_hbm, rm_hbm, rt_hbm, out_hbm, idx_m, idx_t, rows, acc,
                 sem_l, sem_s):
    c = lax.axis_index("c")
    s = lax.axis_index("s")
    wid = s * NC + c
    base = wid * EPW_K
    ntot = NCH_M + 1

    zvec = jnp.zeros((16,), jnp.float32)
    stage = rows.at[0]

    def zrow(i, carry):
        for k in range(Z // 16):
            stage[i, pl.ds(k * 16, 16)] = zvec
        return carry

    lax.fori_loop(0, SCH, zrow, 0)
    for q in range(NPT // SCH):
        pltpu.sync_copy(stage, acc.at[pl.ds(s * NPT + q * SCH, SCH)])
    pltpu.sync_copy(
        stage.at[pl.ds(0, NPT - (NPT // SCH) * SCH)],
        acc.at[pl.ds(s * NPT + (NPT // SCH) * SCH, NPT - (NPT // SCH) * SCH)],
    )
    pltpu.sync_copy(rm_hbm.at[wid], idx_m)
    pltpu.sync_copy(rt_hbm.at[wid], idx_t)
    plsc.subcore_barrier()

    def _is_tail(j):
        return isinstance(j, int) and j == NCH_M

    def l_copy(j, b):
        if _is_tail(j):
            return pltpu.make_async_copy(
                upd_hbm.at[pl.ds(base + NCH_M * SCH, TAIL)],
                rows.at[b, pl.ds(0, TAIL)], sem_l.at[b])
        return pltpu.make_async_copy(
            upd_hbm.at[pl.ds(base + j * SCH, SCH)], rows.at[b], sem_l.at[b])

    def s_copy(j, b):
        if _is_tail(j):
            return pltpu.make_async_copy(rows.at[b, pl.ds(0, TAIL)],
                                         acc.at[idx_t.at[0]], sem_s.at[b])
        return pltpu.make_async_copy(rows.at[b], acc.at[idx_m.at[j]],
                                     sem_s.at[b])

    l_copy(0, 0).start()

    def chunk_step(j, nj_static=None):
        b = j % SBUF if isinstance(j, int) else lax.rem(j, SBUF)
        l_copy(j, b).wait()
        nj = j + 1
        nb = nj % SBUF if isinstance(nj, int) else lax.rem(nj, SBUF)
        if isinstance(j, int):
            if nj < ntot:
                if nj >= SBUF:
                    s_copy(nj - SBUF, nb).wait()
                l_copy(nj, nb).start()
        else:
            @pl.when(nj >= SBUF)
            def _():
                s_copy(nj - SBUF, nb).wait()

            l_copy(nj, nb).start()
        s_copy(j, b).start(add=True)

    def body(j, carry):
        chunk_step(j)
        return carry

    lax.fori_loop(0, NCH_M - 1, body, 0)
    chunk_step(NCH_M - 1)
    chunk_step(NCH_M)
    for t in range(SBUF):
        k = ntot - SBUF + t
        s_copy(k, k % SBUF).wait()
    plsc.subcore_barrier()

    for q in range(NPT // SCH):
        off = s * NPT + q * SCH
        pltpu.sync_copy(acc.at[pl.ds(off, SCH)], stage)
        pltpu.sync_copy(stage, out_hbm.at[c].at[pl.ds(off, SCH)])
    rem = NPT - (NPT // SCH) * SCH
    off = s * NPT + (NPT // SCH) * SCH
    pltpu.sync_copy(acc.at[pl.ds(off, rem)], stage.at[pl.ds(0, rem)])
    pltpu.sync_copy(stage.at[pl.ds(0, rem)], out_hbm.at[c].at[pl.ds(off, rem)])


def _proj_body(nf_ref, w1a_ref, w1b_ref, b1_ref, ps_ref, pr_ref):
    nf = nf_ref[...]
    ps_ref[...] = (
        jnp.dot(nf, w1a_ref[...], preferred_element_type=jnp.float32) + b1_ref[...]
    )
    pr_ref[...] = jnp.dot(nf, w1b_ref[...], preferred_element_type=jnp.float32)


_proj = pl.pallas_call(
    _proj_body,
    grid=(N_NODES // NBLK,),
    in_specs=[
        pl.BlockSpec((NBLK, Z), lambda i: (i, 0)),
        pl.BlockSpec((Z, H), lambda i: (0, 0)),
        pl.BlockSpec((Z, H), lambda i: (0, 0)),
        pl.BlockSpec((1, H), lambda i: (0, 0)),
    ],
    out_specs=[
        pl.BlockSpec((NBLK, H), lambda i: (i, 0)),
        pl.BlockSpec((NBLK, H), lambda i: (i, 0)),
    ],
    out_shape=[
        jax.ShapeDtypeStruct((N_NODES, H), jnp.float32),
        jax.ShapeDtypeStruct((N_NODES, H), jnp.float32),
    ],
)


def _edge_body(g_ref, e_ref, w1c, w2, b2, w3, b3, newin_ref, upd_ref, new_ref):
    del newin_ref
    e = e_ref[...]
    h1 = jnp.maximum(
        g_ref[...] + jnp.dot(e, w1c[...], preferred_element_type=jnp.float32), 0.0
    )
    h2 = jnp.maximum(
        jnp.dot(h1, w2[...], preferred_element_type=jnp.float32) + b2[...], 0.0
    )
    upd = jnp.dot(h2, w3[...], preferred_element_type=jnp.float32) + b3[...]
    upd_ref[...] = upd
    new_ref[...] = upd + e


def _make_edge_mlp(off_blocks):
    return pl.pallas_call(
        _edge_body,
        grid=(NE_K // EBLK,),
        in_specs=[
            pl.BlockSpec((EBLK, H), lambda i: (i, 0)),
            pl.BlockSpec((EBLK, Z), lambda i: (i + off_blocks, 0)),
            pl.BlockSpec((Z, H), lambda i: (0, 0)),
            pl.BlockSpec((H, H), lambda i: (0, 0)),
            pl.BlockSpec((1, H), lambda i: (0, 0)),
            pl.BlockSpec((H, Z), lambda i: (0, 0)),
            pl.BlockSpec((1, Z), lambda i: (0, 0)),
            pl.BlockSpec((8, Z), lambda i: (0, 0)),
        ],
        out_specs=[
            pl.BlockSpec((EBLK, Z), lambda i: (i, 0)),
            pl.BlockSpec((EBLK, Z), lambda i: (i + off_blocks, 0)),
        ],
        out_shape=[
            jax.ShapeDtypeStruct((NE_K, Z), jnp.float32),
            jax.ShapeDtypeStruct((N_EDGES, Z), jnp.float32),
        ],
        input_output_aliases={7: 1},
    )


_edge_mlps = [_make_edge_mlp(k * (NE_K // EBLK)) for k in range(K)]


def _node_body(nf_ref, p0, p1, p2, p3, w1a, w1b, b1, w2, b2, w3, b3, out_ref):
    nf = nf_ref[...]
    agg = (p0[...] + p1[...]) + (p2[...] + p3[...])
    h1 = jnp.maximum(
        jnp.dot(nf, w1a[...], preferred_element_type=jnp.float32)
        + jnp.dot(agg, w1b[...], preferred_element_type=jnp.float32)
        + b1[...],
        0.0,
    )
    h2 = jnp.maximum(
        jnp.dot(h1, w2[...], preferred_element_type=jnp.float32) + b2[...], 0.0
    )
    out_ref[...] = (
        jnp.dot(h2, w3[...], preferred_element_type=jnp.float32) + b3[...] + nf
    )


_node_mlp = pl.pallas_call(
    _node_body,
    grid=(N_NODES // NBLK,),
    in_specs=[
        pl.BlockSpec((NBLK, Z), lambda i: (i, 0)),
        pl.BlockSpec((NBLK, Z), lambda i: (i, 0)),
        pl.BlockSpec((NBLK, Z), lambda i: (i, 0)),
        pl.BlockSpec((NBLK, Z), lambda i: (i, 0)),
        pl.BlockSpec((NBLK, Z), lambda i: (i, 0)),
        pl.BlockSpec((Z, H), lambda i: (0, 0)),
        pl.BlockSpec((Z, H), lambda i: (0, 0)),
        pl.BlockSpec((1, H), lambda i: (0, 0)),
        pl.BlockSpec((H, H), lambda i: (0, 0)),
        pl.BlockSpec((1, H), lambda i: (0, 0)),
        pl.BlockSpec((H, Z), lambda i: (0, 0)),
        pl.BlockSpec((1, Z), lambda i: (0, 0)),
    ],
    out_specs=pl.BlockSpec((NBLK, Z), lambda i: (i, 0)),
    out_shape=jax.ShapeDtypeStruct((N_NODES, Z), jnp.float32),
)


def kernel(node_features, edge_features, senders, receivers,
           eW1, eb1, eW2, eb2, eW3, eb3,
           nW1, nb1, nW2, nb2, nW3, nb3):
    s32 = senders.astype(jnp.int32).reshape(K, NW, NSUP, SUBT, SUBC)
    r32g = receivers.astype(jnp.int32).reshape(K, NW, NSUP, SUBT, SUBC)
    r32w = receivers.astype(jnp.int32).reshape(K, NW, EPW_K)
    r32m = r32w[:, :, :NCH_M * SCH].reshape(K, NW, NCH_M, SCH)
    r32t = r32w[:, :, NCH_M * SCH:].reshape(K, NW, 1, TAIL)

    ps, pr = _proj(node_features, eW1[:Z], eW1[Z:2 * Z], eb1.reshape(1, H))

    w1c = eW1[2 * Z:]
    b2r = eb2.reshape(1, H)
    b3r = eb3.reshape(1, Z)

    g0, new_buf = _gather0(ps, pr, s32[0], r32g[0])
    upd0, new_buf = _edge_mlps[0](
        g0, edge_features, w1c, eW2, b2r, eW3, b3r, new_buf
    )
    g1 = _gather1(ps, pr, s32[1], r32g[1])
    parts0 = _scatter_add(upd0, r32m[0], r32t[0])
    upd1, new_edge = _edge_mlps[1](
        g1, edge_features, w1c, eW2, b2r, eW3, b3r, new_buf
    )
    parts1 = _scatter_add(upd1, r32m[1], r32t[1])

    new_node = _node_mlp(
        node_features,
        parts0[0, :N_NODES], parts0[1, :N_NODES],
        parts1[0, :N_NODES], parts1[1, :N_NODES],
        nW1[:Z], nW1[Z:], nb1.reshape(1, H), nW2, nb2.reshape(1, H), nW3,
        nb3.reshape(1, Z),
    )
    return new_node, new_edge
